# baseline (speedup 1.0000x reference)
import functools
import math

import jax
import jax.numpy as jnp
import numpy as np
from jax.experimental import pallas as pl
from jax.experimental.pallas import tpu as pltpu

_B, _P, _NF, _K = 16, 4096, 3, 64
_S1 = math.ceil(0.2 * _P)
_S2 = math.ceil(0.25 * _S1)


def _mlp_fwd(h, params, mask=None):
    for p in params:
        if len(p) == 4:
            W, b, gamma, beta = p
            h = h @ W + b
            if mask is None:
                mean = jnp.mean(h, axis=0)
                var = jnp.mean((h - mean) ** 2, axis=0)
            else:
                m = mask.astype(h.dtype)[:, None]
                cnt = jnp.maximum(jnp.sum(m), 1.0)
                mean = jnp.sum(h * m, axis=0) / cnt
                var = jnp.sum(((h - mean) ** 2) * m, axis=0) / cnt
            h = (h - mean) / jnp.sqrt(var + 1e-5) * gamma + beta
            h = jnp.where(h >= 0, h, 0.01 * h)
        else:
            W, b = p
            h = h @ W + b
    return h


def _fps(pos, S):
    def fps_one(p):
        d = jnp.sum((p - p[0]) ** 2, axis=1)
        idx0 = jnp.zeros((S,), jnp.int32)

        def body(i, carry):
            idx, d = carry
            nxt = jnp.argmax(d).astype(jnp.int32)
            idx = idx.at[i].set(nxt)
            d = jnp.minimum(d, jnp.sum((p - p[nxt]) ** 2, axis=1))
            return (idx, d)

        idx, _ = jax.lax.fori_loop(1, S, body, (idx0, d))
        return idx

    return jax.vmap(fps_one)(pos)


def _radius_neighbors(pos_all, pos_q, r, k):
    d2 = (jnp.sum(pos_q ** 2, -1)[:, :, None] + jnp.sum(pos_all ** 2, -1)[:, None, :]
          - 2.0 * jnp.einsum('bsd,bpd->bsp', pos_q, pos_all))
    d2 = jnp.maximum(d2, 0.0)
    neg, idx = jax.lax.top_k(-d2, k)
    mask = (-neg) <= r * r
    return idx, mask


_gx = jax.vmap(lambda a, i: a[i])


def _sa_module(x, pos, S, r, params):
    b, p, _ = pos.shape
    idx = _fps(pos, S)
    pos_q = _gx(pos, idx)
    nidx, mask = _radius_neighbors(pos, pos_q, r, _K)
    pos_j = _gx(pos, nidx)
    x_j = _gx(x, nidx)
    rel = pos_j - pos_q[:, :, None, :]
    msg = jnp.concatenate([x_j, rel], axis=-1)
    h = _mlp_fwd(msg.reshape(b * S * _K, -1), params, mask=mask.reshape(-1))
    h = h.reshape(b, S, _K, -1)
    h = jnp.where(mask[..., None], h, -jnp.inf)
    out = jnp.max(h, axis=2)
    return out, pos_q


def _head_body(h0_ref, w1_ref, b1_ref, g1_ref, be1_ref, w2_ref, b2_ref,
               g2_ref, be2_ref, w3_ref, b3_ref, w4_ref, b4_ref, g4_ref,
               be4_ref, w5_ref, b5_ref, g5_ref, be5_ref, w6_ref, b6_ref,
               out_ref):
    def bn_act(h, gamma, beta):
        mean = jnp.mean(h, axis=0)
        var = jnp.mean((h - mean) ** 2, axis=0)
        h = (h - mean) / jnp.sqrt(var + 1e-5) * gamma + beta
        return jnp.where(h >= 0, h, 0.01 * h)

    h = h0_ref[...]
    h = jnp.dot(h, w1_ref[...], preferred_element_type=jnp.float32) + b1_ref[...]
    h = bn_act(h, g1_ref[...], be1_ref[...])
    h = jnp.dot(h, w2_ref[...], preferred_element_type=jnp.float32) + b2_ref[...]
    h = bn_act(h, g2_ref[...], be2_ref[...])
    h = jnp.dot(h, w3_ref[...], preferred_element_type=jnp.float32) + b3_ref[...]
    g = jnp.max(h.reshape(_B, _S2, 1024), axis=1)
    g = jnp.dot(g, w4_ref[...], preferred_element_type=jnp.float32) + b4_ref[...]
    g = bn_act(g, g4_ref[...], be4_ref[...])
    g = jnp.dot(g, w5_ref[...], preferred_element_type=jnp.float32) + b5_ref[...]
    g = bn_act(g, g5_ref[...], be5_ref[...])
    g = jnp.dot(g, w6_ref[...], preferred_element_type=jnp.float32) + b6_ref[...]
    out_ref[...] = g


def _head(x2, pos2, p_sa3, p_mlp):
    h0 = jnp.concatenate([x2, pos2], axis=-1).reshape(_B * _S2, 256 + 3)
    (w1, b1, g1, be1), (w2, b2, g2, be2), (w3, b3) = p_sa3
    (w4, b4, g4, be4), (w5, b5, g5, be5), (w6, b6) = p_mlp
    return pl.pallas_call(
        _head_body,
        out_shape=jax.ShapeDtypeStruct((_B, 1), jnp.float32),
    )(h0, w1, b1, g1, be1, w2, b2, g2, be2, w3, b3,
      w4, b4, g4, be4, w5, b5, g5, be5, w6, b6)


def kernel(x, pos, batch, params):
    xr = x.reshape(_B, _P, _NF)
    pr = pos.reshape(_B, _P, 3)
    x1, pos1 = _sa_module(xr, pr, _S1, 0.1, params['sa1'])
    x2, pos2 = _sa_module(x1, pos1, _S2, 0.5, params['sa2'])
    return _head(x2, pos2, params['sa3'], params['mlp'])


# speedup vs baseline: 10.2792x; 10.2792x over previous
import functools
import math

import jax
import jax.numpy as jnp
import numpy as np
from jax import lax
from jax.experimental import pallas as pl
from jax.experimental.pallas import tpu as pltpu
from jax.experimental.pallas import tpu_sc as plsc

_B, _P, _NF, _K = 16, 4096, 3, 64
_S1 = math.ceil(0.2 * _P)
_S2 = math.ceil(0.25 * _S1)
_L1 = 832
_L2 = 256
_INF = np.float32(np.inf)


def _fps_level(px, py, pz, n_valid, S, acc_lanes):
    Bc, L = px.shape
    lane = jax.lax.broadcasted_iota(jnp.int32, (Bc, L), 1)
    alane = jax.lax.broadcasted_iota(jnp.int32, (Bc, acc_lanes), 1)
    valid = lane < n_valid
    x0, y0, z0 = px[:, 0:1], py[:, 0:1], pz[:, 0:1]
    dx, dy, dz = px - x0, py - y0, pz - z0
    d = (dx * dx + dy * dy) + dz * dz
    d = jnp.where(valid, d, -1.0)
    ix = jnp.zeros((Bc, acc_lanes), jnp.int32)
    ax = jnp.where(alane == 0, x0, 0.0)
    ay = jnp.where(alane == 0, y0, 0.0)
    az = jnp.where(alane == 0, z0, 0.0)

    def body(i, carry):
        d, ix, ax, ay, az = carry
        m = jnp.max(d, axis=1, keepdims=True)
        cand = jnp.where(d == m, lane, L)
        nxt = jnp.min(cand, axis=1, keepdims=True)
        sel = lane == nxt
        qx = jnp.max(jnp.where(sel, px, -_INF), axis=1, keepdims=True)
        qy = jnp.max(jnp.where(sel, py, -_INF), axis=1, keepdims=True)
        qz = jnp.max(jnp.where(sel, pz, -_INF), axis=1, keepdims=True)
        hit = alane == i
        ix = jnp.where(hit, nxt, ix)
        ax = jnp.where(hit, qx, ax)
        ay = jnp.where(hit, qy, ay)
        az = jnp.where(hit, qz, az)
        ddx, ddy, ddz = px - qx, py - qy, pz - qz
        dn = (ddx * ddx + ddy * ddy) + ddz * ddz
        d = jnp.minimum(d, jnp.where(valid, dn, -1.0))
        return (d, ix, ax, ay, az)

    _, ix, ax, ay, az = jax.lax.fori_loop(1, S, body, (d, ix, ax, ay, az))
    return ix, ax, ay, az


def _fps_body(px_ref, py_ref, pz_ref,
              ix1_ref, ax1_ref, ay1_ref, az1_ref,
              ix2_ref, ax2_ref, ay2_ref, az2_ref):
    px, py, pz = px_ref[...], py_ref[...], pz_ref[...]
    ix1, ax1, ay1, az1 = _fps_level(px, py, pz, _P, _S1, _L1)
    ix1_ref[...], ax1_ref[...], ay1_ref[...], az1_ref[...] = ix1, ax1, ay1, az1
    ix2, ax2, ay2, az2 = _fps_level(ax1, ay1, az1, _S1, _S2, _L2)
    ix2_ref[...], ax2_ref[...], ay2_ref[...], az2_ref[...] = ix2, ax2, ay2, az2


def _fps_call(px, py, pz):
    f32, i32 = jnp.float32, jnp.int32
    outs = pl.pallas_call(
        _fps_body,
        out_shape=(
            jax.ShapeDtypeStruct((_B, _L1), i32),
            jax.ShapeDtypeStruct((_B, _L1), f32),
            jax.ShapeDtypeStruct((_B, _L1), f32),
            jax.ShapeDtypeStruct((_B, _L1), f32),
            jax.ShapeDtypeStruct((_B, _L2), i32),
            jax.ShapeDtypeStruct((_B, _L2), f32),
            jax.ShapeDtypeStruct((_B, _L2), f32),
            jax.ShapeDtypeStruct((_B, _L2), f32),
        ),
    )(px, py, pz)
    return outs


def _n2_body(p1x_ref, p1y_ref, p1z_ref, q2x_ref, q2y_ref, q2z_ref,
             nidx_ref, mval_ref):
    px, py, pz = p1x_ref[...][0], p1y_ref[...][0], p1z_ref[...][0]
    qx = jnp.transpose(q2x_ref[...][0][:, :208])
    qy = jnp.transpose(q2y_ref[...][0][:, :208])
    qz = jnp.transpose(q2z_ref[...][0][:, :208])
    q3 = jnp.concatenate([qx, qy, qz], axis=1)
    p3 = jnp.concatenate([px, py, pz], axis=0)
    cross = jnp.dot(q3, p3, preferred_element_type=jnp.float32)
    sqq = (qx * qx + qy * qy) + qz * qz
    sqp = (px * px + py * py) + pz * pz
    d2 = (sqq + sqp) - 2.0 * cross
    d2 = jnp.maximum(d2, 0.0)
    lane = jax.lax.broadcasted_iota(jnp.int32, (208, _L1), 1)
    d2 = jnp.where(lane < _S1, d2, _INF)
    slot = jax.lax.broadcasted_iota(jnp.int32, (1, _K), 1)
    accI = jnp.zeros((208, _K), jnp.int32)
    accV = jnp.zeros((208, _K), jnp.float32)

    def body(s, carry):
        d2, accI, accV = carry
        m = jnp.min(d2, axis=1, keepdims=True)
        cand = jnp.where(d2 == m, lane, _L1)
        j = jnp.min(cand, axis=1, keepdims=True)
        hit = slot == s
        accI = jnp.where(hit, j, accI)
        accV = jnp.where(hit, m, accV)
        d2 = jnp.where(lane == j, _INF, d2)
        return (d2, accI, accV)

    _, accI, accV = jax.lax.fori_loop(0, _K, body, (d2, accI, accV))
    nidx_ref[...] = accI[None]
    mval_ref[...] = accV[None]


def _n2_call(p1x, p1y, p1z, q2x, q2y, q2z):
    nidx, mval = pl.pallas_call(
        _n2_body,
        grid=(_B,),
        in_specs=[pl.BlockSpec((1, 1, _L1), lambda b: (b, 0, 0))] * 3
                 + [pl.BlockSpec((1, 1, _L2), lambda b: (b, 0, 0))] * 3,
        out_specs=(pl.BlockSpec((1, 208, _K), lambda b: (b, 0, 0)),
                   pl.BlockSpec((1, 208, _K), lambda b: (b, 0, 0))),
        out_shape=(jax.ShapeDtypeStruct((_B, 208, _K), jnp.int32),
                   jax.ShapeDtypeStruct((_B, 208, _K), jnp.float32)),
    )(p1x[:, None], p1y[:, None], p1z[:, None],
      q2x[:, None], q2y[:, None], q2z[:, None])
    return nidx[:, :_S2], mval[:, :_S2]


def _sc_gather(table, idx, CH):
    E = idx.shape[0]
    D = table.shape[1]
    NW = 32
    per_w = E // NW
    n_ch = per_w // CH
    assert per_w % CH == 0 and CH % 128 == 0 and E % NW == 0
    nsub = CH // 128
    mesh = plsc.VectorSubcoreMesh(core_axis_name="c", subcore_axis_name="s")

    @functools.partial(
        pl.kernel, mesh=mesh,
        out_type=jax.ShapeDtypeStruct((E, D), jnp.float32),
        compiler_params=pltpu.CompilerParams(use_tc_tiling_on_sc=False),
        scratch_types=[
            pltpu.VMEM((CH,), jnp.int32),
            pltpu.VMEM((CH, D), jnp.float32),
            pltpu.SemaphoreType.DMA,
        ],
    )
    def gk(table_hbm, idx_hbm, out_hbm, idx_v, rows_v, sem):
        wid = lax.axis_index("s") * 2 + lax.axis_index("c")
        base = wid * per_w

        def body(j, carry):
            off = base + j * CH
            pltpu.sync_copy(idx_hbm.at[pl.ds(off, CH)], idx_v)
            cps = [pltpu.async_copy(
                       table_hbm.at[idx_v.at[pl.ds(t * 128, 128)]],
                       rows_v.at[pl.ds(t * 128, 128)], sem)
                   for t in range(nsub)]
            for cp in cps:
                cp.wait()
            pltpu.sync_copy(rows_v, out_hbm.at[pl.ds(off, CH)])
            return carry

        lax.fori_loop(0, n_ch, body, 0)

    return gk(table, idx)


_NQT = 416
_SLOTS = 96


def _sc_select1(pxf, pyf, pzf, qxf, qyf, qzf):
    i32, f32 = jnp.int32, jnp.float32
    mesh = plsc.VectorSubcoreMesh(core_axis_name="c", subcore_axis_name="s")
    r2 = np.float32(0.1) * np.float32(0.1)

    @functools.partial(
        pl.kernel, mesh=mesh,
        out_type=jax.ShapeDtypeStruct((_B * 832 * _SLOTS,), i32),
        compiler_params=pltpu.CompilerParams(use_tc_tiling_on_sc=False,
                                             needs_layout_passes=False),
        scratch_types=[
            pltpu.VMEM((_P,), f32),
            pltpu.VMEM((_P,), f32),
            pltpu.VMEM((_P,), f32),
            pltpu.VMEM((_NQT,), f32),
            pltpu.VMEM((_NQT,), f32),
            pltpu.VMEM((_NQT,), f32),
            pltpu.VMEM((_NQT * _SLOTS,), i32),
        ],
    )
    def nk(px_h, py_h, pz_h, qx_h, qy_h, qz_h, out_h,
           pxv, pyv, pzv, qxv, qyv, qzv, buf):
        wid = lax.axis_index("s") * 2 + lax.axis_index("c")
        cloud = wid // 2
        soff = (wid % 2) * _NQT
        pltpu.sync_copy(px_h.at[pl.ds(cloud * _P, _P)], pxv)
        pltpu.sync_copy(py_h.at[pl.ds(cloud * _P, _P)], pyv)
        pltpu.sync_copy(pz_h.at[pl.ds(cloud * _P, _P)], pzv)
        qbase = cloud * 832 + soff
        pltpu.sync_copy(qx_h.at[pl.ds(qbase, _NQT)], qxv)
        pltpu.sync_copy(qy_h.at[pl.ds(qbase, _NQT)], qyv)
        pltpu.sync_copy(qz_h.at[pl.ds(qbase, _NQT)], qzv)

        sent = jnp.full((16,), _SENT1, i32)

        def fill(i, c):
            buf[pl.ds(i * 16, 16)] = sent
            return c

        lax.fori_loop(0, _NQT * _SLOTS // 16, fill, 0)

        iota16 = lax.broadcasted_iota(i32, (16,), 0)
        gbase = cloud * _P

        def per_query(q, c):
            qg = (q // 16) * 16
            sel16 = iota16 == (q % 16)
            zf = jnp.zeros((16,), f32)

            def pick(ref):
                v = jnp.sum(jnp.where(sel16, ref[pl.ds(qg, 16)], 0.0), axis=0)
                return zf + v

            qxs, qys, qzs = pick(qxv), pick(qyv), pick(qzv)

            def per_chunk(ci, cur):
                pxc = pxv[pl.ds(ci * 16, 16)]
                pyc = pyv[pl.ds(ci * 16, 16)]
                pzc = pzv[pl.ds(ci * 16, 16)]
                dx, dy, dz = pxc - qxs, pyc - qys, pzc - qzs
                d2 = (dx * dx + dy * dy) + dz * dz
                msk = d2 <= r2
                vals = (gbase + ci * 16) + iota16
                plsc.store_compressed(buf.at[pl.ds(q * _SLOTS + cur, 16)],
                                      vals, mask=msk)
                cnt = jnp.sum(msk.astype(i32), axis=0)
                return jnp.minimum(cur + cnt, _SLOTS - 16)

            lax.fori_loop(0, _P // 16, per_chunk, jnp.int32(0))
            return c

        lax.fori_loop(0, _NQT, per_query, 0)
        pltpu.sync_copy(buf, out_h.at[pl.ds(qbase * _SLOTS, _NQT * _SLOTS)])

    return nk(pxf, pyf, pzf, qxf, qyf, qzf)


_EB = 8192
_QB = 128


def _bn_from_partials(s1_ref, s2_ref, cn_ref):
    cnt = jnp.maximum(jnp.sum(cn_ref[...]), 1.0)
    mean = jnp.sum(s1_ref[...], axis=0) / cnt
    var = jnp.sum(s2_ref[...], axis=0) / cnt - mean * mean
    return mean, var


def _msg_h1(G_ref, Qe_ref, w1_ref, b1_ref):
    G = G_ref[...]
    qe = jnp.broadcast_to(Qe_ref[...][:, None, :], (_QB, _K, G.shape[1]))
    msg = G - qe.reshape(G.shape)
    return jnp.dot(msg, w1_ref[...], preferred_element_type=jnp.float32) + b1_ref[...]


def _bn_act(h, mean, var, g_ref, be_ref):
    h = (h - mean) / jnp.sqrt(var + 1e-5) * g_ref[...] + be_ref[...]
    return jnp.where(h >= 0, h, 0.01 * h)


def _eA_body(G_ref, Qe_ref, emk_ref, w1_ref, b1_ref, s1_ref, s2_ref, cn_ref):
    h1 = _msg_h1(G_ref, Qe_ref, w1_ref, b1_ref)
    m = emk_ref[...]
    hm = h1 * m
    s1_ref[...] = jnp.sum(hm, axis=0, keepdims=True)[None]
    s2_ref[...] = jnp.sum(hm * h1, axis=0, keepdims=True)[None]
    cn_ref[...] = jnp.sum(m, axis=0, keepdims=True)[None]


def _eB_body(G_ref, Qe_ref, emk_ref, w1_ref, b1_ref, g1_ref, be1_ref,
             s1a_ref, s2a_ref, cna_ref, w2_ref, b2_ref, s1_ref, s2_ref):
    mean1, var1 = _bn_from_partials(s1a_ref, s2a_ref, cna_ref)
    h1 = _msg_h1(G_ref, Qe_ref, w1_ref, b1_ref)
    a1 = _bn_act(h1, mean1, var1, g1_ref, be1_ref)
    h2 = jnp.dot(a1, w2_ref[...], preferred_element_type=jnp.float32) + b2_ref[...]
    m = emk_ref[...]
    hm = h2 * m
    s1_ref[...] = jnp.sum(hm, axis=0, keepdims=True)[None]
    s2_ref[...] = jnp.sum(hm * h2, axis=0, keepdims=True)[None]


def _eC_body(G_ref, Qe_ref, emk_ref, w1_ref, b1_ref, g1_ref, be1_ref,
             s1a_ref, s2a_ref, cna_ref, w2_ref, b2_ref, g2_ref, be2_ref,
             s1b_ref, s2b_ref, w3_ref, b3_ref, out_ref):
    mean1, var1 = _bn_from_partials(s1a_ref, s2a_ref, cna_ref)
    mean2, var2 = _bn_from_partials(s1b_ref, s2b_ref, cna_ref)
    h1 = _msg_h1(G_ref, Qe_ref, w1_ref, b1_ref)
    a1 = _bn_act(h1, mean1, var1, g1_ref, be1_ref)
    h2 = jnp.dot(a1, w2_ref[...], preferred_element_type=jnp.float32) + b2_ref[...]
    a2 = _bn_act(h2, mean2, var2, g2_ref, be2_ref)
    h3 = jnp.dot(a2, w3_ref[...], preferred_element_type=jnp.float32) + b3_ref[...]
    m = emk_ref[...]
    neg = jnp.where(m > 0, h3, -_INF)
    out_ref[...] = jnp.max(neg.reshape(_QB, _K, h3.shape[1]), axis=1)


def _edge_mlp_pallas(G, Qe, emk, params, H, Dout):
    (w1, b1, g1, be1), (w2, b2, g2, be2), (w3, b3) = params
    Epad, Dinp = G.shape
    Qpad = Qe.shape[0]
    nblk = Epad // _EB
    f32 = jnp.float32

    def row(v):
        return v.reshape(1, -1)

    w1p = jnp.zeros((Dinp, H), f32).at[:w1.shape[0]].set(w1)
    full = lambda a: pl.BlockSpec(a.shape, lambda i: tuple(0 for _ in a.shape))
    gspec = pl.BlockSpec((_EB, Dinp), lambda i: (i, 0))
    qspec = pl.BlockSpec((_QB, Dinp), lambda i: (i, 0))
    mspec = pl.BlockSpec((_EB, 1), lambda i: (i, 0))
    pspec = lambda w: pl.BlockSpec((1, 1, w), lambda i: (i, 0, 0))

    b1r, g1r, be1r = row(b1), row(g1), row(be1)
    b2r, g2r, be2r = row(b2), row(g2), row(be2)
    b3r = row(b3)

    s1a, s2a, cna = pl.pallas_call(
        _eA_body, grid=(nblk,),
        in_specs=[gspec, qspec, mspec, full(w1p), full(b1r)],
        out_specs=(pspec(H), pspec(H), pspec(1)),
        out_shape=(jax.ShapeDtypeStruct((nblk, 1, H), f32),
                   jax.ShapeDtypeStruct((nblk, 1, H), f32),
                   jax.ShapeDtypeStruct((nblk, 1, 1), f32)),
    )(G, Qe, emk, w1p, b1r)

    s1b, s2b = pl.pallas_call(
        _eB_body, grid=(nblk,),
        in_specs=[gspec, qspec, mspec, full(w1p), full(b1r), full(g1r),
                  full(be1r), full(s1a), full(s2a), full(cna), full(w2),
                  full(b2r)],
        out_specs=(pspec(H), pspec(H)),
        out_shape=(jax.ShapeDtypeStruct((nblk, 1, H), f32),
                   jax.ShapeDtypeStruct((nblk, 1, H), f32)),
    )(G, Qe, emk, w1p, b1r, g1r, be1r, s1a, s2a, cna, w2, b2r)

    out = pl.pallas_call(
        _eC_body, grid=(nblk,),
        in_specs=[gspec, qspec, mspec, full(w1p), full(b1r), full(g1r),
                  full(be1r), full(s1a), full(s2a), full(cna), full(w2),
                  full(b2r), full(g2r), full(be2r), full(s1b), full(s2b),
                  full(w3), full(b3r)],
        out_specs=pl.BlockSpec((_QB, Dout), lambda i: (i, 0)),
        out_shape=jax.ShapeDtypeStruct((Qpad, Dout), f32),
    )(G, Qe, emk, w1p, b1r, g1r, be1r, s1a, s2a, cna, w2, b2r, g2r, be2r,
      s1b, s2b, w3, b3r)
    return out


def _head_body(h0_ref, w1_ref, b1_ref, g1_ref, be1_ref, w2_ref, b2_ref,
               g2_ref, be2_ref, w3_ref, b3_ref, w4_ref, b4_ref, g4_ref,
               be4_ref, w5_ref, b5_ref, g5_ref, be5_ref, w6_ref, b6_ref,
               out_ref):
    def bn_act(h, gamma, beta):
        mean = jnp.mean(h, axis=0)
        var = jnp.mean((h - mean) ** 2, axis=0)
        h = (h - mean) / jnp.sqrt(var + 1e-5) * gamma + beta
        return jnp.where(h >= 0, h, 0.01 * h)

    h = h0_ref[...]
    h = jnp.dot(h, w1_ref[...], preferred_element_type=jnp.float32) + b1_ref[...]
    h = bn_act(h, g1_ref[...], be1_ref[...])
    h = jnp.dot(h, w2_ref[...], preferred_element_type=jnp.float32) + b2_ref[...]
    h = bn_act(h, g2_ref[...], be2_ref[...])
    h = jnp.dot(h, w3_ref[...], preferred_element_type=jnp.float32) + b3_ref[...]
    g = jnp.max(h.reshape(_B, _S2, 1024), axis=1)
    g = jnp.dot(g, w4_ref[...], preferred_element_type=jnp.float32) + b4_ref[...]
    g = bn_act(g, g4_ref[...], be4_ref[...])
    g = jnp.dot(g, w5_ref[...], preferred_element_type=jnp.float32) + b5_ref[...]
    g = bn_act(g, g5_ref[...], be5_ref[...])
    g = jnp.dot(g, w6_ref[...], preferred_element_type=jnp.float32) + b6_ref[...]
    out_ref[...] = g


def _head(x2, pos2, p_sa3, p_mlp):
    h0 = jnp.concatenate([x2, pos2], axis=-1).reshape(_B * _S2, 256 + 3)
    (w1, b1, g1, be1), (w2, b2, g2, be2), (w3, b3) = p_sa3
    (w4, b4, g4, be4), (w5, b5, g5, be5), (w6, b6) = p_mlp
    return pl.pallas_call(
        _head_body,
        out_shape=jax.ShapeDtypeStruct((_B, 1), jnp.float32),
    )(h0, w1, b1, g1, be1, w2, b2, g2, be2, w3, b3,
      w4, b4, g4, be4, w5, b5, g5, be5, w6, b6)


_SENT1 = _B * _P
_Q1 = 13312
_Q2 = 3328


def kernel(x, pos, batch, params):
    f32, i32 = jnp.float32, jnp.int32
    xr = x.reshape(_B, _P, _NF)
    pr = pos.reshape(_B, _P, 3)
    px, py, pz = pr[..., 0], pr[..., 1], pr[..., 2]

    (_ix1, ax1, ay1, az1, _ix2, ax2, ay2, az2) = _fps_call(px, py, pz)
    pos1 = jnp.stack([ax1[:, :_S1], ay1[:, :_S1], az1[:, :_S1]], axis=-1)
    pos2 = jnp.stack([ax2[:, :_S2], ay2[:, :_S2], az2[:, :_S2]], axis=-1)

    sel = _sc_select1(px.reshape(-1), py.reshape(-1), pz.reshape(-1),
                      ax1.reshape(-1), ay1.reshape(-1), az1.reshape(-1))
    gidx1 = sel.reshape(_B, 832, _SLOTS)[:, :_S1, :_K]
    gidx1f = jnp.concatenate(
        [gidx1.reshape(_B * _S1, _K),
         jnp.full((_Q1 - _B * _S1, _K), _SENT1, i32)], axis=0)
    emk1 = (gidx1f != _SENT1).astype(f32).reshape(-1, 1)
    table1 = jnp.concatenate(
        [xr.reshape(_B * _P, 3), pr.reshape(_B * _P, 3),
         jnp.zeros((_B * _P, 10), f32)], axis=1)
    table1 = jnp.concatenate([table1, jnp.zeros((8, 16), f32)], axis=0)
    G1 = _sc_gather(table1, gidx1f.reshape(-1), 2048)
    Qe1 = jnp.zeros((_Q1, 16), f32).at[:_B * _S1, 3:6].set(
        pos1.reshape(_B * _S1, 3))
    x1f = _edge_mlp_pallas(G1, Qe1, emk1, params['sa1'], 64, 128)
    x1 = x1f[:_B * _S1].reshape(_B, _S1, 128)

    nidx2, mval2 = _n2_call(ax1, ay1, az1, ax2, ay2, az2)
    mask2 = mval2 <= jnp.float32(0.25)
    gidx2 = nidx2 + (jnp.arange(_B, dtype=i32) * _S1)[:, None, None]
    gidx2f = jnp.concatenate(
        [gidx2.reshape(_B * _S2, _K),
         jnp.zeros((_Q2 - _B * _S2, _K), i32)], axis=0)
    emk2 = jnp.concatenate(
        [mask2.reshape(_B * _S2, _K).astype(f32),
         jnp.zeros((_Q2 - _B * _S2, _K), f32)], axis=0).reshape(-1, 1)
    table2 = jnp.concatenate(
        [x1.reshape(_B * _S1, 128), pos1.reshape(_B * _S1, 3),
         jnp.zeros((_B * _S1, 13), f32)], axis=1)
    G2 = _sc_gather(table2, gidx2f.reshape(-1), 512)
    Qe2 = jnp.zeros((_Q2, 144), f32).at[:_B * _S2, 128:131].set(
        pos2.reshape(_B * _S2, 3))
    x2f = _edge_mlp_pallas(G2, Qe2, emk2, params['sa2'], 128, 256)
    x2 = x2f[:_B * _S2].reshape(_B, _S2, 256)

    return _head(x2, pos2, params['sa3'], params['mlp'])


# speedup vs baseline: 11.7251x; 1.1407x over previous
import functools
import math

import jax
import jax.numpy as jnp
import numpy as np
from jax import lax
from jax.experimental import pallas as pl
from jax.experimental.pallas import tpu as pltpu
from jax.experimental.pallas import tpu_sc as plsc

_B, _P, _NF, _K = 16, 4096, 3, 64
_S1 = math.ceil(0.2 * _P)
_S2 = math.ceil(0.25 * _S1)
_L1 = 832
_L2 = 256
_INF = np.float32(np.inf)


def _fps_level(px, py, pz, n_valid, S, acc_lanes):
    Bc, L = px.shape
    lane = jax.lax.broadcasted_iota(jnp.int32, (Bc, L), 1)
    alane = jax.lax.broadcasted_iota(jnp.int32, (Bc, acc_lanes), 1)
    valid = lane < n_valid
    x0, y0, z0 = px[:, 0:1], py[:, 0:1], pz[:, 0:1]
    dx, dy, dz = px - x0, py - y0, pz - z0
    d = (dx * dx + dy * dy) + dz * dz
    d = jnp.where(valid, d, -1.0)
    ix = jnp.zeros((Bc, acc_lanes), jnp.int32)
    ax = jnp.where(alane == 0, x0, 0.0)
    ay = jnp.where(alane == 0, y0, 0.0)
    az = jnp.where(alane == 0, z0, 0.0)

    def body(i, carry):
        d, ix, ax, ay, az = carry
        m = jnp.max(d, axis=1, keepdims=True)
        cand = jnp.where(d == m, lane, L)
        nxt = jnp.min(cand, axis=1, keepdims=True)
        sel = lane == nxt
        qx = jnp.max(jnp.where(sel, px, -_INF), axis=1, keepdims=True)
        qy = jnp.max(jnp.where(sel, py, -_INF), axis=1, keepdims=True)
        qz = jnp.max(jnp.where(sel, pz, -_INF), axis=1, keepdims=True)
        hit = alane == i
        ix = jnp.where(hit, nxt, ix)
        ax = jnp.where(hit, qx, ax)
        ay = jnp.where(hit, qy, ay)
        az = jnp.where(hit, qz, az)
        ddx, ddy, ddz = px - qx, py - qy, pz - qz
        dn = (ddx * ddx + ddy * ddy) + ddz * ddz
        d = jnp.minimum(d, jnp.where(valid, dn, -1.0))
        return (d, ix, ax, ay, az)

    _, ix, ax, ay, az = jax.lax.fori_loop(1, S, body, (d, ix, ax, ay, az))
    return ix, ax, ay, az


def _fps_body(px_ref, py_ref, pz_ref,
              ix1_ref, ax1_ref, ay1_ref, az1_ref,
              ix2_ref, ax2_ref, ay2_ref, az2_ref):
    px, py, pz = px_ref[...], py_ref[...], pz_ref[...]
    ix1, ax1, ay1, az1 = _fps_level(px, py, pz, _P, _S1, _L1)
    ix1_ref[...], ax1_ref[...], ay1_ref[...], az1_ref[...] = ix1, ax1, ay1, az1
    ix2, ax2, ay2, az2 = _fps_level(ax1, ay1, az1, _S1, _S2, _L2)
    ix2_ref[...], ax2_ref[...], ay2_ref[...], az2_ref[...] = ix2, ax2, ay2, az2


def _fps_call(px, py, pz):
    f32, i32 = jnp.float32, jnp.int32
    outs = pl.pallas_call(
        _fps_body,
        out_shape=(
            jax.ShapeDtypeStruct((_B, _L1), i32),
            jax.ShapeDtypeStruct((_B, _L1), f32),
            jax.ShapeDtypeStruct((_B, _L1), f32),
            jax.ShapeDtypeStruct((_B, _L1), f32),
            jax.ShapeDtypeStruct((_B, _L2), i32),
            jax.ShapeDtypeStruct((_B, _L2), f32),
            jax.ShapeDtypeStruct((_B, _L2), f32),
            jax.ShapeDtypeStruct((_B, _L2), f32),
        ),
    )(px, py, pz)
    return outs


def _n2_body(p1x_ref, p1y_ref, p1z_ref, q2x_ref, q2y_ref, q2z_ref,
             nidx_ref, mval_ref):
    px, py, pz = p1x_ref[...][0], p1y_ref[...][0], p1z_ref[...][0]
    qx = jnp.transpose(q2x_ref[...][0][:, :208])
    qy = jnp.transpose(q2y_ref[...][0][:, :208])
    qz = jnp.transpose(q2z_ref[...][0][:, :208])
    q3 = jnp.concatenate([qx, qy, qz], axis=1)
    p3 = jnp.concatenate([px, py, pz], axis=0)
    cross = jnp.dot(q3, p3, preferred_element_type=jnp.float32)
    sqq = (qx * qx + qy * qy) + qz * qz
    sqp = (px * px + py * py) + pz * pz
    d2 = (sqq + sqp) - 2.0 * cross
    d2 = jnp.maximum(d2, 0.0)
    lane = jax.lax.broadcasted_iota(jnp.int32, (208, _L1), 1)
    d2 = jnp.where(lane < _S1, d2, _INF)
    slot = jax.lax.broadcasted_iota(jnp.int32, (1, _K), 1)
    accI = jnp.zeros((208, _K), jnp.int32)
    accV = jnp.zeros((208, _K), jnp.float32)

    def body(s, carry):
        d2, accI, accV = carry
        m = jnp.min(d2, axis=1, keepdims=True)
        cand = jnp.where(d2 == m, lane, _L1)
        j = jnp.min(cand, axis=1, keepdims=True)
        hit = slot == s
        accI = jnp.where(hit, j, accI)
        accV = jnp.where(hit, m, accV)
        d2 = jnp.where(lane == j, _INF, d2)
        return (d2, accI, accV)

    _, accI, accV = jax.lax.fori_loop(0, _K, body, (d2, accI, accV))
    nidx_ref[...] = accI[None]
    mval_ref[...] = accV[None]


def _n2_call(p1x, p1y, p1z, q2x, q2y, q2z):
    nidx, mval = pl.pallas_call(
        _n2_body,
        grid=(_B,),
        in_specs=[pl.BlockSpec((1, 1, _L1), lambda b: (b, 0, 0))] * 3
                 + [pl.BlockSpec((1, 1, _L2), lambda b: (b, 0, 0))] * 3,
        out_specs=(pl.BlockSpec((1, 208, _K), lambda b: (b, 0, 0)),
                   pl.BlockSpec((1, 208, _K), lambda b: (b, 0, 0))),
        out_shape=(jax.ShapeDtypeStruct((_B, 208, _K), jnp.int32),
                   jax.ShapeDtypeStruct((_B, 208, _K), jnp.float32)),
    )(p1x[:, None], p1y[:, None], p1z[:, None],
      q2x[:, None], q2y[:, None], q2z[:, None])
    return nidx[:, :_S2], mval[:, :_S2]


def _sc_gather(table, idx, CH):
    E = idx.shape[0]
    D = table.shape[1]
    NW = 32
    per_w = E // NW
    n_ch = per_w // CH
    assert per_w % CH == 0 and CH % 128 == 0 and E % NW == 0
    nsub = CH // 128
    mesh = plsc.VectorSubcoreMesh(core_axis_name="c", subcore_axis_name="s")

    @functools.partial(
        pl.kernel, mesh=mesh,
        out_type=jax.ShapeDtypeStruct((E, D), jnp.float32),
        compiler_params=pltpu.CompilerParams(use_tc_tiling_on_sc=False),
        scratch_types=[
            pltpu.VMEM((CH,), jnp.int32),
            pltpu.VMEM((CH, D), jnp.float32),
            pltpu.SemaphoreType.DMA,
        ],
    )
    def gk(table_hbm, idx_hbm, out_hbm, idx_v, rows_v, sem):
        wid = lax.axis_index("s") * 2 + lax.axis_index("c")
        base = wid * per_w

        def body(j, carry):
            off = base + j * CH
            pltpu.sync_copy(idx_hbm.at[pl.ds(off, CH)], idx_v)
            pltpu.async_copy(table_hbm.at[idx_v], rows_v, sem).wait()
            pltpu.sync_copy(rows_v, out_hbm.at[pl.ds(off, CH)])
            return carry

        lax.fori_loop(0, n_ch, body, 0)

    return gk(table, idx)


_NQT = 416
_SLOTS = 96


def _sc_select1(pxf, pyf, pzf, qxf, qyf, qzf):
    i32, f32 = jnp.int32, jnp.float32
    mesh = plsc.VectorSubcoreMesh(core_axis_name="c", subcore_axis_name="s")
    r2 = np.float32(0.1) * np.float32(0.1)

    @functools.partial(
        pl.kernel, mesh=mesh,
        out_type=jax.ShapeDtypeStruct((_B * 832 * _SLOTS,), i32),
        compiler_params=pltpu.CompilerParams(use_tc_tiling_on_sc=False,
                                             needs_layout_passes=False),
        scratch_types=[
            pltpu.VMEM((_P,), f32),
            pltpu.VMEM((_P,), f32),
            pltpu.VMEM((_P,), f32),
            pltpu.VMEM((_NQT,), f32),
            pltpu.VMEM((_NQT,), f32),
            pltpu.VMEM((_NQT,), f32),
            pltpu.VMEM((_NQT * _SLOTS,), i32),
        ],
    )
    def nk(px_h, py_h, pz_h, qx_h, qy_h, qz_h, out_h,
           pxv, pyv, pzv, qxv, qyv, qzv, buf):
        wid = lax.axis_index("s") * 2 + lax.axis_index("c")
        cloud = wid // 2
        soff = (wid % 2) * _NQT
        pltpu.sync_copy(px_h.at[pl.ds(cloud * _P, _P)], pxv)
        pltpu.sync_copy(py_h.at[pl.ds(cloud * _P, _P)], pyv)
        pltpu.sync_copy(pz_h.at[pl.ds(cloud * _P, _P)], pzv)
        qbase = cloud * 832 + soff
        pltpu.sync_copy(qx_h.at[pl.ds(qbase, _NQT)], qxv)
        pltpu.sync_copy(qy_h.at[pl.ds(qbase, _NQT)], qyv)
        pltpu.sync_copy(qz_h.at[pl.ds(qbase, _NQT)], qzv)

        sent = jnp.full((16,), _SENT1, i32)

        def fill(i, c):
            buf[pl.ds(i * 16, 16)] = sent
            return c

        lax.fori_loop(0, _NQT * _SLOTS // 16, fill, 0)

        iota16 = lax.broadcasted_iota(i32, (16,), 0)
        gbase = cloud * _P

        def per_qgroup(qq, c):
            zf = jnp.zeros((16,), f32)

            def pick(ref, q):
                sel16 = iota16 == (q % 16)
                v = jnp.sum(jnp.where(sel16, ref[pl.ds((q // 16) * 16, 16)],
                                      0.0), axis=0)
                return zf + v

            qs = []
            for k in range(4):
                q = qq * 4 + k
                qs.append((q, pick(qxv, q), pick(qyv, q), pick(qzv, q)))

            def per_chunk(ci, curs):
                pxc = pxv[pl.ds(ci * 16, 16)]
                pyc = pyv[pl.ds(ci * 16, 16)]
                pzc = pzv[pl.ds(ci * 16, 16)]
                vals = (gbase + ci * 16) + iota16
                out = []
                for k in range(4):
                    q, qxs, qys, qzs = qs[k]
                    dx, dy, dz = pxc - qxs, pyc - qys, pzc - qzs
                    d2 = (dx * dx + dy * dy) + dz * dz
                    msk = d2 <= r2
                    plsc.store_compressed(
                        buf.at[pl.ds(q * _SLOTS + curs[k], 16)], vals, mask=msk)
                    cnt = jnp.sum(msk.astype(i32), axis=0)
                    out.append(jnp.minimum(curs[k] + cnt, _SLOTS - 16))
                return tuple(out)

            lax.fori_loop(0, _P // 16, per_chunk,
                          (jnp.int32(0),) * 4)
            return c

        lax.fori_loop(0, _NQT // 4, per_qgroup, 0)
        pltpu.sync_copy(buf, out_h.at[pl.ds(qbase * _SLOTS, _NQT * _SLOTS)])

    return nk(pxf, pyf, pzf, qxf, qyf, qzf)


_EB = 8192
_QB = 128


def _bn_from_partials(s1_ref, s2_ref, cn_ref):
    cnt = jnp.maximum(jnp.sum(cn_ref[...]), 1.0)
    mean = jnp.sum(s1_ref[...], axis=0) / cnt
    var = jnp.sum(s2_ref[...], axis=0) / cnt - mean * mean
    return mean, var


def _msg_h1(G_ref, Qe_ref, w1_ref, b1_ref):
    G = G_ref[...]
    qe = jnp.broadcast_to(Qe_ref[...][:, None, :], (_QB, _K, G.shape[1]))
    msg = G - qe.reshape(G.shape)
    return jnp.dot(msg, w1_ref[...], preferred_element_type=jnp.float32) + b1_ref[...]


def _bn_act(h, mean, var, g_ref, be_ref):
    h = (h - mean) / jnp.sqrt(var + 1e-5) * g_ref[...] + be_ref[...]
    return jnp.where(h >= 0, h, 0.01 * h)


def _eA_body(G_ref, Qe_ref, emk_ref, w1_ref, b1_ref, s1_ref, s2_ref, cn_ref):
    h1 = _msg_h1(G_ref, Qe_ref, w1_ref, b1_ref)
    m = emk_ref[...]
    hm = h1 * m
    s1_ref[...] = jnp.sum(hm, axis=0, keepdims=True)[None]
    s2_ref[...] = jnp.sum(hm * h1, axis=0, keepdims=True)[None]
    cn_ref[...] = jnp.sum(m, axis=0, keepdims=True)[None]


def _eB_body(G_ref, Qe_ref, emk_ref, w1_ref, b1_ref, g1_ref, be1_ref,
             s1a_ref, s2a_ref, cna_ref, w2_ref, b2_ref, s1_ref, s2_ref):
    mean1, var1 = _bn_from_partials(s1a_ref, s2a_ref, cna_ref)
    h1 = _msg_h1(G_ref, Qe_ref, w1_ref, b1_ref)
    a1 = _bn_act(h1, mean1, var1, g1_ref, be1_ref)
    h2 = jnp.dot(a1, w2_ref[...], preferred_element_type=jnp.float32) + b2_ref[...]
    m = emk_ref[...]
    hm = h2 * m
    s1_ref[...] = jnp.sum(hm, axis=0, keepdims=True)[None]
    s2_ref[...] = jnp.sum(hm * h2, axis=0, keepdims=True)[None]


def _eC_body(G_ref, Qe_ref, emk_ref, w1_ref, b1_ref, g1_ref, be1_ref,
             s1a_ref, s2a_ref, cna_ref, w2_ref, b2_ref, g2_ref, be2_ref,
             s1b_ref, s2b_ref, w3_ref, b3_ref, out_ref):
    mean1, var1 = _bn_from_partials(s1a_ref, s2a_ref, cna_ref)
    mean2, var2 = _bn_from_partials(s1b_ref, s2b_ref, cna_ref)
    h1 = _msg_h1(G_ref, Qe_ref, w1_ref, b1_ref)
    a1 = _bn_act(h1, mean1, var1, g1_ref, be1_ref)
    h2 = jnp.dot(a1, w2_ref[...], preferred_element_type=jnp.float32) + b2_ref[...]
    a2 = _bn_act(h2, mean2, var2, g2_ref, be2_ref)
    h3 = jnp.dot(a2, w3_ref[...], preferred_element_type=jnp.float32) + b3_ref[...]
    m = emk_ref[...]
    neg = jnp.where(m > 0, h3, -_INF)
    out_ref[...] = jnp.max(neg.reshape(_QB, _K, h3.shape[1]), axis=1)


def _edge_mlp_pallas(G, Qe, emk, params, H, Dout):
    (w1, b1, g1, be1), (w2, b2, g2, be2), (w3, b3) = params
    Epad, Dinp = G.shape
    Qpad = Qe.shape[0]
    nblk = Epad // _EB
    f32 = jnp.float32

    def row(v):
        return v.reshape(1, -1)

    w1p = jnp.zeros((Dinp, H), f32).at[:w1.shape[0]].set(w1)
    full = lambda a: pl.BlockSpec(a.shape, lambda i: tuple(0 for _ in a.shape))
    gspec = pl.BlockSpec((_EB, Dinp), lambda i: (i, 0))
    qspec = pl.BlockSpec((_QB, Dinp), lambda i: (i, 0))
    mspec = pl.BlockSpec((_EB, 1), lambda i: (i, 0))
    pspec = lambda w: pl.BlockSpec((1, 1, w), lambda i: (i, 0, 0))

    b1r, g1r, be1r = row(b1), row(g1), row(be1)
    b2r, g2r, be2r = row(b2), row(g2), row(be2)
    b3r = row(b3)

    s1a, s2a, cna = pl.pallas_call(
        _eA_body, grid=(nblk,),
        in_specs=[gspec, qspec, mspec, full(w1p), full(b1r)],
        out_specs=(pspec(H), pspec(H), pspec(1)),
        out_shape=(jax.ShapeDtypeStruct((nblk, 1, H), f32),
                   jax.ShapeDtypeStruct((nblk, 1, H), f32),
                   jax.ShapeDtypeStruct((nblk, 1, 1), f32)),
    )(G, Qe, emk, w1p, b1r)

    s1b, s2b = pl.pallas_call(
        _eB_body, grid=(nblk,),
        in_specs=[gspec, qspec, mspec, full(w1p), full(b1r), full(g1r),
                  full(be1r), full(s1a), full(s2a), full(cna), full(w2),
                  full(b2r)],
        out_specs=(pspec(H), pspec(H)),
        out_shape=(jax.ShapeDtypeStruct((nblk, 1, H), f32),
                   jax.ShapeDtypeStruct((nblk, 1, H), f32)),
    )(G, Qe, emk, w1p, b1r, g1r, be1r, s1a, s2a, cna, w2, b2r)

    out = pl.pallas_call(
        _eC_body, grid=(nblk,),
        in_specs=[gspec, qspec, mspec, full(w1p), full(b1r), full(g1r),
                  full(be1r), full(s1a), full(s2a), full(cna), full(w2),
                  full(b2r), full(g2r), full(be2r), full(s1b), full(s2b),
                  full(w3), full(b3r)],
        out_specs=pl.BlockSpec((_QB, Dout), lambda i: (i, 0)),
        out_shape=jax.ShapeDtypeStruct((Qpad, Dout), f32),
    )(G, Qe, emk, w1p, b1r, g1r, be1r, s1a, s2a, cna, w2, b2r, g2r, be2r,
      s1b, s2b, w3, b3r)
    return out


def _head_body(h0_ref, w1_ref, b1_ref, g1_ref, be1_ref, w2_ref, b2_ref,
               g2_ref, be2_ref, w3_ref, b3_ref, w4_ref, b4_ref, g4_ref,
               be4_ref, w5_ref, b5_ref, g5_ref, be5_ref, w6_ref, b6_ref,
               out_ref):
    def bn_act(h, gamma, beta):
        mean = jnp.mean(h, axis=0)
        var = jnp.mean((h - mean) ** 2, axis=0)
        h = (h - mean) / jnp.sqrt(var + 1e-5) * gamma + beta
        return jnp.where(h >= 0, h, 0.01 * h)

    h = h0_ref[...]
    h = jnp.dot(h, w1_ref[...], preferred_element_type=jnp.float32) + b1_ref[...]
    h = bn_act(h, g1_ref[...], be1_ref[...])
    h = jnp.dot(h, w2_ref[...], preferred_element_type=jnp.float32) + b2_ref[...]
    h = bn_act(h, g2_ref[...], be2_ref[...])
    h = jnp.dot(h, w3_ref[...], preferred_element_type=jnp.float32) + b3_ref[...]
    g = jnp.max(h.reshape(_B, _S2, 1024), axis=1)
    g = jnp.dot(g, w4_ref[...], preferred_element_type=jnp.float32) + b4_ref[...]
    g = bn_act(g, g4_ref[...], be4_ref[...])
    g = jnp.dot(g, w5_ref[...], preferred_element_type=jnp.float32) + b5_ref[...]
    g = bn_act(g, g5_ref[...], be5_ref[...])
    g = jnp.dot(g, w6_ref[...], preferred_element_type=jnp.float32) + b6_ref[...]
    out_ref[...] = g


def _head(x2, pos2, p_sa3, p_mlp):
    h0 = jnp.concatenate([x2, pos2], axis=-1).reshape(_B * _S2, 256 + 3)
    (w1, b1, g1, be1), (w2, b2, g2, be2), (w3, b3) = p_sa3
    (w4, b4, g4, be4), (w5, b5, g5, be5), (w6, b6) = p_mlp
    return pl.pallas_call(
        _head_body,
        out_shape=jax.ShapeDtypeStruct((_B, 1), jnp.float32),
    )(h0, w1, b1, g1, be1, w2, b2, g2, be2, w3, b3,
      w4, b4, g4, be4, w5, b5, g5, be5, w6, b6)


_SENT1 = _B * _P
_Q1 = 13312
_Q2 = 3328


def kernel(x, pos, batch, params):
    f32, i32 = jnp.float32, jnp.int32
    xr = x.reshape(_B, _P, _NF)
    pr = pos.reshape(_B, _P, 3)
    px, py, pz = pr[..., 0], pr[..., 1], pr[..., 2]

    (_ix1, ax1, ay1, az1, _ix2, ax2, ay2, az2) = _fps_call(px, py, pz)
    pos1 = jnp.stack([ax1[:, :_S1], ay1[:, :_S1], az1[:, :_S1]], axis=-1)
    pos2 = jnp.stack([ax2[:, :_S2], ay2[:, :_S2], az2[:, :_S2]], axis=-1)

    sel = _sc_select1(px.reshape(-1), py.reshape(-1), pz.reshape(-1),
                      ax1.reshape(-1), ay1.reshape(-1), az1.reshape(-1))
    gidx1 = sel.reshape(_B, 832, _SLOTS)[:, :_S1, :_K]
    gidx1f = jnp.concatenate(
        [gidx1.reshape(_B * _S1, _K),
         jnp.full((_Q1 - _B * _S1, _K), _SENT1, i32)], axis=0)
    emk1 = (gidx1f != _SENT1).astype(f32).reshape(-1, 1)
    table1 = jnp.concatenate(
        [xr.reshape(_B * _P, 3), pr.reshape(_B * _P, 3),
         jnp.zeros((_B * _P, 10), f32)], axis=1)
    table1 = jnp.concatenate([table1, jnp.zeros((8, 16), f32)], axis=0)
    G1 = _sc_gather(table1, gidx1f.reshape(-1), 2048)
    Qe1 = jnp.zeros((_Q1, 16), f32).at[:_B * _S1, 3:6].set(
        pos1.reshape(_B * _S1, 3))
    x1f = _edge_mlp_pallas(G1, Qe1, emk1, params['sa1'], 64, 128)
    x1 = x1f[:_B * _S1].reshape(_B, _S1, 128)

    nidx2, mval2 = _n2_call(ax1, ay1, az1, ax2, ay2, az2)
    mask2 = mval2 <= jnp.float32(0.25)
    gidx2 = nidx2 + (jnp.arange(_B, dtype=i32) * _S1)[:, None, None]
    gidx2f = jnp.concatenate(
        [gidx2.reshape(_B * _S2, _K),
         jnp.zeros((_Q2 - _B * _S2, _K), i32)], axis=0)
    emk2 = jnp.concatenate(
        [mask2.reshape(_B * _S2, _K).astype(f32),
         jnp.zeros((_Q2 - _B * _S2, _K), f32)], axis=0).reshape(-1, 1)
    table2 = jnp.concatenate(
        [x1.reshape(_B * _S1, 128), pos1.reshape(_B * _S1, 3),
         jnp.zeros((_B * _S1, 13), f32)], axis=1)
    G2 = _sc_gather(table2, gidx2f.reshape(-1), 512)
    Qe2 = jnp.zeros((_Q2, 144), f32).at[:_B * _S2, 128:131].set(
        pos2.reshape(_B * _S2, 3))
    x2f = _edge_mlp_pallas(G2, Qe2, emk2, params['sa2'], 128, 256)
    x2 = x2f[:_B * _S2].reshape(_B, _S2, 256)

    return _head(x2, pos2, params['sa3'], params['mlp'])


# speedup vs baseline: 20.8248x; 1.7761x over previous
import functools
import math

import jax
import jax.numpy as jnp
import numpy as np
from jax import lax
from jax.experimental import pallas as pl
from jax.experimental.pallas import tpu as pltpu
from jax.experimental.pallas import tpu_sc as plsc

_B, _P, _NF, _K = 16, 4096, 3, 64
_S1 = math.ceil(0.2 * _P)
_S2 = math.ceil(0.25 * _S1)
_L1 = 832
_L2 = 256
_INF = np.float32(np.inf)


def _fps_level(px, py, pz, n_valid, S, acc_lanes):
    Bc, L = px.shape
    lane = jax.lax.broadcasted_iota(jnp.int32, (Bc, L), 1)
    alane = jax.lax.broadcasted_iota(jnp.int32, (Bc, acc_lanes), 1)
    valid = lane < n_valid
    x0, y0, z0 = px[:, 0:1], py[:, 0:1], pz[:, 0:1]
    dx, dy, dz = px - x0, py - y0, pz - z0
    d = (dx * dx + dy * dy) + dz * dz
    d = jnp.where(valid, d, -1.0)
    ix = jnp.zeros((Bc, acc_lanes), jnp.int32)
    ax = jnp.where(alane == 0, x0, 0.0)
    ay = jnp.where(alane == 0, y0, 0.0)
    az = jnp.where(alane == 0, z0, 0.0)

    def body(i, carry):
        d, ix, ax, ay, az = carry
        m = jnp.max(d, axis=1, keepdims=True)
        cand = jnp.where(d == m, lane, L)
        nxt = jnp.min(cand, axis=1, keepdims=True)
        sel = lane == nxt
        qx = jnp.max(jnp.where(sel, px, -_INF), axis=1, keepdims=True)
        qy = jnp.max(jnp.where(sel, py, -_INF), axis=1, keepdims=True)
        qz = jnp.max(jnp.where(sel, pz, -_INF), axis=1, keepdims=True)
        hit = alane == i
        ix = jnp.where(hit, nxt, ix)
        ax = jnp.where(hit, qx, ax)
        ay = jnp.where(hit, qy, ay)
        az = jnp.where(hit, qz, az)
        ddx, ddy, ddz = px - qx, py - qy, pz - qz
        dn = (ddx * ddx + ddy * ddy) + ddz * ddz
        d = jnp.minimum(d, jnp.where(valid, dn, -1.0))
        return (d, ix, ax, ay, az)

    _, ix, ax, ay, az = jax.lax.fori_loop(1, S, body, (d, ix, ax, ay, az))
    return ix, ax, ay, az


def _fps_body(px_ref, py_ref, pz_ref,
              ix1_ref, ax1_ref, ay1_ref, az1_ref,
              ix2_ref, ax2_ref, ay2_ref, az2_ref):
    px, py, pz = px_ref[...], py_ref[...], pz_ref[...]
    ix1, ax1, ay1, az1 = _fps_level(px, py, pz, _P, _S1, _L1)
    ix1_ref[...], ax1_ref[...], ay1_ref[...], az1_ref[...] = ix1, ax1, ay1, az1
    ix2, ax2, ay2, az2 = _fps_level(ax1, ay1, az1, _S1, _S2, _L2)
    ix2_ref[...], ax2_ref[...], ay2_ref[...], az2_ref[...] = ix2, ax2, ay2, az2


def _fps_call(px, py, pz):
    f32, i32 = jnp.float32, jnp.int32
    outs = pl.pallas_call(
        _fps_body,
        out_shape=(
            jax.ShapeDtypeStruct((_B, _L1), i32),
            jax.ShapeDtypeStruct((_B, _L1), f32),
            jax.ShapeDtypeStruct((_B, _L1), f32),
            jax.ShapeDtypeStruct((_B, _L1), f32),
            jax.ShapeDtypeStruct((_B, _L2), i32),
            jax.ShapeDtypeStruct((_B, _L2), f32),
            jax.ShapeDtypeStruct((_B, _L2), f32),
            jax.ShapeDtypeStruct((_B, _L2), f32),
        ),
    )(px, py, pz)
    return outs


def _n2t_body(p1x_ref, p1y_ref, p1z_ref, q2x_ref, q2y_ref, q2z_ref, thr_ref):
    px, py, pz = p1x_ref[...][0], p1y_ref[...][0], p1z_ref[...][0]
    qx = jnp.transpose(q2x_ref[...][0][:, :208])
    qy = jnp.transpose(q2y_ref[...][0][:, :208])
    qz = jnp.transpose(q2z_ref[...][0][:, :208])
    q3 = jnp.concatenate([qx, qy, qz], axis=1)
    p3 = jnp.concatenate([px, py, pz], axis=0)
    cross = jnp.dot(q3, p3, preferred_element_type=jnp.float32)
    sqq = (qx * qx + qy * qy) + qz * qz
    sqp = (px * px + py * py) + pz * pz
    d2 = (sqq + sqp) - 2.0 * cross
    d2 = jnp.maximum(d2, 0.0)
    lane = jax.lax.broadcasted_iota(jnp.int32, (208, _L1), 1)
    d2 = jnp.where(lane < _S1, d2, _INF)
    lo = jnp.zeros((208, 1), jnp.float32)
    hi = jnp.full((208, 1), 3.0, jnp.float32)

    def it(i, carry):
        lo, hi = carry
        t = (lo + hi) * 0.5
        c = jnp.sum((d2 <= t).astype(jnp.float32), axis=1, keepdims=True)
        ge = c >= float(_K)
        return (jnp.where(ge, lo, t), jnp.where(ge, t, hi))

    lo, hi = jax.lax.fori_loop(0, 44, it, (lo, hi))
    thr = jnp.minimum(hi, 0.25)
    thr_ref[...] = jnp.transpose(thr)[None]


def _n2t_call(p1x, p1y, p1z, q2x, q2y, q2z):
    thr = pl.pallas_call(
        _n2t_body,
        grid=(_B,),
        in_specs=[pl.BlockSpec((1, 1, _L1), lambda b: (b, 0, 0))] * 3
                 + [pl.BlockSpec((1, 1, _L2), lambda b: (b, 0, 0))] * 3,
        out_specs=pl.BlockSpec((1, 1, 208), lambda b: (b, 0, 0)),
        out_shape=jax.ShapeDtypeStruct((_B, 1, 208), jnp.float32),
    )(p1x[:, None], p1y[:, None], p1z[:, None],
      q2x[:, None], q2y[:, None], q2z[:, None])
    return thr.reshape(_B, 208)


def _sc_gather(table, idx, CH):
    E = idx.shape[0]
    D = table.shape[1]
    NW = 32
    per_w = E // NW
    n_ch = per_w // CH
    assert per_w % CH == 0 and CH % 128 == 0 and E % NW == 0
    nsub = CH // 128
    mesh = plsc.VectorSubcoreMesh(core_axis_name="c", subcore_axis_name="s")

    @functools.partial(
        pl.kernel, mesh=mesh,
        out_type=jax.ShapeDtypeStruct((E, D), jnp.float32),
        compiler_params=pltpu.CompilerParams(use_tc_tiling_on_sc=False),
        scratch_types=[
            pltpu.VMEM((CH,), jnp.int32),
            pltpu.VMEM((CH, D), jnp.float32),
            pltpu.SemaphoreType.DMA,
        ],
    )
    def gk(table_hbm, idx_hbm, out_hbm, idx_v, rows_v, sem):
        wid = lax.axis_index("s") * 2 + lax.axis_index("c")
        base = wid * per_w

        def body(j, carry):
            off = base + j * CH
            pltpu.sync_copy(idx_hbm.at[pl.ds(off, CH)], idx_v)
            pltpu.async_copy(table_hbm.at[idx_v], rows_v, sem).wait()
            pltpu.sync_copy(rows_v, out_hbm.at[pl.ds(off, CH)])
            return carry

        lax.fori_loop(0, n_ch, body, 0)

    return gk(table, idx)


_SLOTS = 96


def _sc_select(pxf, pyf, pzf, qxf, qyf, qzf, thrf, PC, QC, sent):
    i32, f32 = jnp.int32, jnp.float32
    mesh = plsc.VectorSubcoreMesh(core_axis_name="c", subcore_axis_name="s")
    NQT = _B * QC // 32

    @functools.partial(
        pl.kernel, mesh=mesh,
        out_type=jax.ShapeDtypeStruct((_B * QC * _SLOTS,), i32),
        compiler_params=pltpu.CompilerParams(use_tc_tiling_on_sc=False,
                                             needs_layout_passes=False),
        scratch_types=[
            pltpu.VMEM((PC,), f32),
            pltpu.VMEM((PC,), f32),
            pltpu.VMEM((PC,), f32),
            pltpu.VMEM((NQT,), f32),
            pltpu.VMEM((NQT,), f32),
            pltpu.VMEM((NQT,), f32),
            pltpu.VMEM((NQT,), f32),
            pltpu.VMEM((NQT * _SLOTS,), i32),
        ],
    )
    def nk(px_h, py_h, pz_h, qx_h, qy_h, qz_h, th_h, out_h,
           pxv, pyv, pzv, qxv, qyv, qzv, thv, buf):
        wid = lax.axis_index("s") * 2 + lax.axis_index("c")
        cloud = wid // 2
        soff = (wid % 2) * NQT
        pltpu.sync_copy(px_h.at[pl.ds(cloud * PC, PC)], pxv)
        pltpu.sync_copy(py_h.at[pl.ds(cloud * PC, PC)], pyv)
        pltpu.sync_copy(pz_h.at[pl.ds(cloud * PC, PC)], pzv)
        qbase = cloud * QC + soff
        pltpu.sync_copy(qx_h.at[pl.ds(qbase, NQT)], qxv)
        pltpu.sync_copy(qy_h.at[pl.ds(qbase, NQT)], qyv)
        pltpu.sync_copy(qz_h.at[pl.ds(qbase, NQT)], qzv)
        pltpu.sync_copy(th_h.at[pl.ds(qbase, NQT)], thv)

        sentv = jnp.full((16,), sent, i32)

        def fill(i, c):
            buf[pl.ds(i * 16, 16)] = sentv
            return c

        lax.fori_loop(0, NQT * _SLOTS // 16, fill, 0)

        iota16 = lax.broadcasted_iota(i32, (16,), 0)
        gbase = cloud * PC

        def per_qgroup(qq, c):
            zf = jnp.zeros((16,), f32)

            def pick(ref, q):
                sel16 = iota16 == (q % 16)
                v = jnp.sum(jnp.where(sel16, ref[pl.ds((q // 16) * 16, 16)],
                                      0.0), axis=0)
                return zf + v

            qs = []
            for k in range(4):
                q = qq * 4 + k
                qs.append((q, pick(qxv, q), pick(qyv, q), pick(qzv, q),
                           pick(thv, q)))

            def per_chunk(ci, curs):
                pxc = pxv[pl.ds(ci * 16, 16)]
                pyc = pyv[pl.ds(ci * 16, 16)]
                pzc = pzv[pl.ds(ci * 16, 16)]
                vals = (gbase + ci * 16) + iota16
                out = []
                for k in range(4):
                    q, qxs, qys, qzs, ths = qs[k]
                    dx, dy, dz = pxc - qxs, pyc - qys, pzc - qzs
                    d2 = (dx * dx + dy * dy) + dz * dz
                    msk = d2 <= ths
                    plsc.store_compressed(
                        buf.at[pl.ds(q * _SLOTS + curs[k], 16)], vals, mask=msk)
                    cnt = jnp.sum(msk.astype(i32), axis=0)
                    out.append(jnp.minimum(curs[k] + cnt, _SLOTS - 16))
                return tuple(out)

            lax.fori_loop(0, PC // 16, per_chunk,
                          (jnp.int32(0),) * 4)
            return c

        lax.fori_loop(0, NQT // 4, per_qgroup, 0)
        pltpu.sync_copy(buf, out_h.at[pl.ds(qbase * _SLOTS, NQT * _SLOTS)])

    return nk(pxf, pyf, pzf, qxf, qyf, qzf, thrf)


_EB = 8192
_QB = 128


def _bn_from_partials(s1_ref, s2_ref, cn_ref):
    cnt = jnp.maximum(jnp.sum(cn_ref[...]), 1.0)
    mean = jnp.sum(s1_ref[...], axis=0) / cnt
    var = jnp.sum(s2_ref[...], axis=0) / cnt - mean * mean
    return mean, var


def _msg_h1(G_ref, Qe_ref, w1_ref, b1_ref):
    G = G_ref[...]
    qe = jnp.broadcast_to(Qe_ref[...][:, None, :], (_QB, _K, G.shape[1]))
    msg = G - qe.reshape(G.shape)
    return jnp.dot(msg, w1_ref[...], preferred_element_type=jnp.float32) + b1_ref[...]


def _bn_act(h, mean, var, g_ref, be_ref):
    h = (h - mean) / jnp.sqrt(var + 1e-5) * g_ref[...] + be_ref[...]
    return jnp.where(h >= 0, h, 0.01 * h)


def _eA_body(G_ref, Qe_ref, emk_ref, w1_ref, b1_ref, s1_ref, s2_ref, cn_ref):
    h1 = _msg_h1(G_ref, Qe_ref, w1_ref, b1_ref)
    m = emk_ref[...]
    hm = h1 * m
    s1_ref[...] = jnp.sum(hm, axis=0, keepdims=True)[None]
    s2_ref[...] = jnp.sum(hm * h1, axis=0, keepdims=True)[None]
    cn_ref[...] = jnp.sum(m, axis=0, keepdims=True)[None]


def _eB_body(G_ref, Qe_ref, emk_ref, w1_ref, b1_ref, g1_ref, be1_ref,
             s1a_ref, s2a_ref, cna_ref, w2_ref, b2_ref, s1_ref, s2_ref):
    mean1, var1 = _bn_from_partials(s1a_ref, s2a_ref, cna_ref)
    h1 = _msg_h1(G_ref, Qe_ref, w1_ref, b1_ref)
    a1 = _bn_act(h1, mean1, var1, g1_ref, be1_ref)
    h2 = jnp.dot(a1, w2_ref[...], preferred_element_type=jnp.float32) + b2_ref[...]
    m = emk_ref[...]
    hm = h2 * m
    s1_ref[...] = jnp.sum(hm, axis=0, keepdims=True)[None]
    s2_ref[...] = jnp.sum(hm * h2, axis=0, keepdims=True)[None]


def _eC_body(G_ref, Qe_ref, emk_ref, w1_ref, b1_ref, g1_ref, be1_ref,
             s1a_ref, s2a_ref, cna_ref, w2_ref, b2_ref, g2_ref, be2_ref,
             s1b_ref, s2b_ref, w3_ref, b3_ref, out_ref):
    mean1, var1 = _bn_from_partials(s1a_ref, s2a_ref, cna_ref)
    mean2, var2 = _bn_from_partials(s1b_ref, s2b_ref, cna_ref)
    h1 = _msg_h1(G_ref, Qe_ref, w1_ref, b1_ref)
    a1 = _bn_act(h1, mean1, var1, g1_ref, be1_ref)
    h2 = jnp.dot(a1, w2_ref[...], preferred_element_type=jnp.float32) + b2_ref[...]
    a2 = _bn_act(h2, mean2, var2, g2_ref, be2_ref)
    h3 = jnp.dot(a2, w3_ref[...], preferred_element_type=jnp.float32) + b3_ref[...]
    m = emk_ref[...]
    neg = jnp.where(m > 0, h3, -_INF)
    out_ref[...] = jnp.max(neg.reshape(_QB, _K, h3.shape[1]), axis=1)


def _edge_mlp_pallas(G, Qe, emk, params, H, Dout):
    (w1, b1, g1, be1), (w2, b2, g2, be2), (w3, b3) = params
    Epad, Dinp = G.shape
    Qpad = Qe.shape[0]
    nblk = Epad // _EB
    f32 = jnp.float32

    def row(v):
        return v.reshape(1, -1)

    w1p = jnp.zeros((Dinp, H), f32).at[:w1.shape[0]].set(w1)
    full = lambda a: pl.BlockSpec(a.shape, lambda i: tuple(0 for _ in a.shape))
    gspec = pl.BlockSpec((_EB, Dinp), lambda i: (i, 0))
    qspec = pl.BlockSpec((_QB, Dinp), lambda i: (i, 0))
    mspec = pl.BlockSpec((_EB, 1), lambda i: (i, 0))
    pspec = lambda w: pl.BlockSpec((1, 1, w), lambda i: (i, 0, 0))

    b1r, g1r, be1r = row(b1), row(g1), row(be1)
    b2r, g2r, be2r = row(b2), row(g2), row(be2)
    b3r = row(b3)

    s1a, s2a, cna = pl.pallas_call(
        _eA_body, grid=(nblk,),
        in_specs=[gspec, qspec, mspec, full(w1p), full(b1r)],
        out_specs=(pspec(H), pspec(H), pspec(1)),
        out_shape=(jax.ShapeDtypeStruct((nblk, 1, H), f32),
                   jax.ShapeDtypeStruct((nblk, 1, H), f32),
                   jax.ShapeDtypeStruct((nblk, 1, 1), f32)),
    )(G, Qe, emk, w1p, b1r)

    s1b, s2b = pl.pallas_call(
        _eB_body, grid=(nblk,),
        in_specs=[gspec, qspec, mspec, full(w1p), full(b1r), full(g1r),
                  full(be1r), full(s1a), full(s2a), full(cna), full(w2),
                  full(b2r)],
        out_specs=(pspec(H), pspec(H)),
        out_shape=(jax.ShapeDtypeStruct((nblk, 1, H), f32),
                   jax.ShapeDtypeStruct((nblk, 1, H), f32)),
    )(G, Qe, emk, w1p, b1r, g1r, be1r, s1a, s2a, cna, w2, b2r)

    out = pl.pallas_call(
        _eC_body, grid=(nblk,),
        in_specs=[gspec, qspec, mspec, full(w1p), full(b1r), full(g1r),
                  full(be1r), full(s1a), full(s2a), full(cna), full(w2),
                  full(b2r), full(g2r), full(be2r), full(s1b), full(s2b),
                  full(w3), full(b3r)],
        out_specs=pl.BlockSpec((_QB, Dout), lambda i: (i, 0)),
        out_shape=jax.ShapeDtypeStruct((Qpad, Dout), f32),
    )(G, Qe, emk, w1p, b1r, g1r, be1r, s1a, s2a, cna, w2, b2r, g2r, be2r,
      s1b, s2b, w3, b3r)
    return out


def _head_body(h0_ref, w1_ref, b1_ref, g1_ref, be1_ref, w2_ref, b2_ref,
               g2_ref, be2_ref, w3_ref, b3_ref, w4_ref, b4_ref, g4_ref,
               be4_ref, w5_ref, b5_ref, g5_ref, be5_ref, w6_ref, b6_ref,
               out_ref):
    def bn_act(h, gamma, beta):
        mean = jnp.mean(h, axis=0)
        var = jnp.mean((h - mean) ** 2, axis=0)
        h = (h - mean) / jnp.sqrt(var + 1e-5) * gamma + beta
        return jnp.where(h >= 0, h, 0.01 * h)

    h = h0_ref[...]
    h = jnp.dot(h, w1_ref[...], preferred_element_type=jnp.float32) + b1_ref[...]
    h = bn_act(h, g1_ref[...], be1_ref[...])
    h = jnp.dot(h, w2_ref[...], preferred_element_type=jnp.float32) + b2_ref[...]
    h = bn_act(h, g2_ref[...], be2_ref[...])
    h = jnp.dot(h, w3_ref[...], preferred_element_type=jnp.float32) + b3_ref[...]
    g = jnp.max(h.reshape(_B, _S2, 1024), axis=1)
    g = jnp.dot(g, w4_ref[...], preferred_element_type=jnp.float32) + b4_ref[...]
    g = bn_act(g, g4_ref[...], be4_ref[...])
    g = jnp.dot(g, w5_ref[...], preferred_element_type=jnp.float32) + b5_ref[...]
    g = bn_act(g, g5_ref[...], be5_ref[...])
    g = jnp.dot(g, w6_ref[...], preferred_element_type=jnp.float32) + b6_ref[...]
    out_ref[...] = g


def _head(x2, pos2, p_sa3, p_mlp):
    h0 = jnp.concatenate([x2, pos2], axis=-1).reshape(_B * _S2, 256 + 3)
    (w1, b1, g1, be1), (w2, b2, g2, be2), (w3, b3) = p_sa3
    (w4, b4, g4, be4), (w5, b5, g5, be5), (w6, b6) = p_mlp
    return pl.pallas_call(
        _head_body,
        out_shape=jax.ShapeDtypeStruct((_B, 1), jnp.float32),
    )(h0, w1, b1, g1, be1, w2, b2, g2, be2, w3, b3,
      w4, b4, g4, be4, w5, b5, g5, be5, w6, b6)


_SENT1 = _B * _P
_SENT2 = _B * _L1
_Q1 = 13312
_Q2 = 3328


def _scatter_sent(gidxf, sent):
    Q, K = gidxf.shape
    eiota = jax.lax.broadcasted_iota(jnp.int32, (Q, K), 1) + \
        jax.lax.broadcasted_iota(jnp.int32, (Q, K), 0) * K
    return jnp.where(gidxf < sent, gidxf, sent + (eiota & 2047))


def kernel(x, pos, batch, params):
    f32, i32 = jnp.float32, jnp.int32
    xr = x.reshape(_B, _P, _NF)
    pr = pos.reshape(_B, _P, 3)
    px, py, pz = pr[..., 0], pr[..., 1], pr[..., 2]

    (_ix1, ax1, ay1, az1, _ix2, ax2, ay2, az2) = _fps_call(px, py, pz)
    pos1 = jnp.stack([ax1[:, :_S1], ay1[:, :_S1], az1[:, :_S1]], axis=-1)
    pos2 = jnp.stack([ax2[:, :_S2], ay2[:, :_S2], az2[:, :_S2]], axis=-1)

    thr1 = jnp.full((_B * _L1,), np.float32(0.1 * 0.1), f32)
    sel = _sc_select(px.reshape(-1), py.reshape(-1), pz.reshape(-1),
                     ax1.reshape(-1), ay1.reshape(-1), az1.reshape(-1),
                     thr1, _P, _L1, _SENT1)
    gidx1 = sel.reshape(_B, _L1, _SLOTS)[:, :_S1, :_K]
    gidx1f = jnp.concatenate(
        [gidx1.reshape(_B * _S1, _K),
         jnp.full((_Q1 - _B * _S1, _K), _SENT1, i32)], axis=0)
    emk1 = (gidx1f < _SENT1).astype(f32).reshape(-1, 1)
    gidx1f = _scatter_sent(gidx1f, _SENT1)
    table1 = jnp.concatenate(
        [xr.reshape(_B * _P, 3), pr.reshape(_B * _P, 3),
         jnp.zeros((_B * _P, 10), f32)], axis=1)
    table1 = jnp.concatenate([table1, jnp.zeros((2048, 16), f32)], axis=0)
    G1 = _sc_gather(table1, gidx1f.reshape(-1), 2048)
    Qe1 = jnp.zeros((_Q1, 16), f32).at[:_B * _S1, 3:6].set(
        pos1.reshape(_B * _S1, 3))
    x1f = _edge_mlp_pallas(G1, Qe1, emk1, params['sa1'], 64, 128)
    x1 = x1f[:_B * _S1].reshape(_B, _S1, 128)

    thr2 = _n2t_call(ax1, ay1, az1, ax2, ay2, az2)
    lanev = (jnp.arange(_L1) < _S1)[None, :]
    p2x = jnp.where(lanev, ax1, np.float32(1e9))
    p2y = jnp.where(lanev, ay1, np.float32(1e9))
    p2z = jnp.where(lanev, az1, np.float32(1e9))
    sel2 = _sc_select(p2x.reshape(-1), p2y.reshape(-1), p2z.reshape(-1),
                      ax2[:, :208].reshape(-1), ay2[:, :208].reshape(-1),
                      az2[:, :208].reshape(-1), thr2.reshape(-1),
                      _L1, 208, _SENT2)
    gidx2 = sel2.reshape(_B, 208, _SLOTS)[:, :_S2, :_K]
    gidx2f = jnp.concatenate(
        [gidx2.reshape(_B * _S2, _K),
         jnp.full((_Q2 - _B * _S2, _K), _SENT2, i32)], axis=0)
    emk2 = (gidx2f < _SENT2).astype(f32).reshape(-1, 1)
    gidx2f = _scatter_sent(gidx2f, _SENT2)
    x1p = jnp.pad(x1, ((0, 0), (0, _L1 - _S1), (0, 0)))
    table2 = jnp.concatenate(
        [x1p.reshape(_B * _L1, 128),
         jnp.stack([ax1, ay1, az1], axis=-1).reshape(_B * _L1, 3),
         jnp.zeros((_B * _L1, 13), f32)], axis=1)
    table2 = jnp.concatenate([table2, jnp.zeros((2048, 144), f32)], axis=0)
    G2 = _sc_gather(table2, gidx2f.reshape(-1), 512)
    Qe2 = jnp.zeros((_Q2, 144), f32).at[:_B * _S2, 128:131].set(
        pos2.reshape(_B * _S2, 3))
    x2f = _edge_mlp_pallas(G2, Qe2, emk2, params['sa2'], 128, 256)
    x2 = x2f[:_B * _S2].reshape(_B, _S2, 256)

    return _head(x2, pos2, params['sa3'], params['mlp'])
